# Initial kernel scaffold; baseline (speedup 1.0000x reference)
import jax
import jax.numpy as jnp
from jax.experimental import pallas as pl


def kernel(x, edge_index, edge_attr, W1, att1, We1, b1, W2, att2, We2, b2, W3, att3, We3, b3):
    raise NotImplementedError("write your pallas kernel here")


# speedup vs baseline: 1.2494x; 1.2494x over previous
import jax
import jax.numpy as jnp
from jax.experimental import pallas as pl

N = 10000
NEG = 0.2


def _final_div_kernel(acc_ref, den_ref, b_ref, out_ref):
    out_ref[...] = acc_ref[...] / den_ref[...] + b_ref[...]


def _gatv2_unnorm(xl, src, dst, ge, att, num_nodes):
    m = xl[src] + xl[dst] + ge
    e = jnp.sum(jax.nn.leaky_relu(m, NEG) * att, axis=-1)
    emax = jax.ops.segment_max(e, dst, num_segments=num_nodes)
    emax = jnp.where(jnp.isfinite(emax), emax, 0.0)
    ee = jnp.exp(e - emax[dst])
    denom = jax.ops.segment_sum(ee, dst, num_segments=num_nodes)
    acc = jax.ops.segment_sum(ee[:, None] * xl[src], dst, num_segments=num_nodes)
    return acc, denom


def kernel(x, edge_index, edge_attr, W1, att1, We1, b1, W2, att2, We2, b2, W3, att3, We3, b3):
    src, dst = edge_index[0], edge_index[1]
    sums = jax.ops.segment_sum(edge_attr, dst, num_segments=N)
    cnt = jax.ops.segment_sum(jnp.ones((edge_attr.shape[0],), edge_attr.dtype), dst, num_segments=N)
    mean = sums / jnp.maximum(cnt, 1.0)[:, None]
    loop = jnp.arange(N, dtype=edge_index.dtype)
    src2 = jnp.concatenate([src, loop])
    dst2 = jnp.concatenate([dst, loop])
    ea2 = jnp.concatenate([edge_attr, mean], axis=0)

    def layer(xin, W, att, We, b):
        xl = xin @ W
        ge = ea2 @ We
        acc, den = _gatv2_unnorm(xl, src2, dst2, ge, att, N)
        out = pl.pallas_call(
            _final_div_kernel,
            out_shape=jax.ShapeDtypeStruct(acc.shape, acc.dtype),
        )(acc, jnp.maximum(den, 1e-16)[:, None], jnp.broadcast_to(b, acc.shape))
        return out

    latent = layer(x, W1, att1, We1, b1)
    node_logits = layer(latent, W2, att2, We2, b2)[:, 0]
    action_logits = layer(latent, W3, att3, We3, b3)
    node_sel = jax.random.categorical(jax.random.key(42), node_logits)
    node_lp = jax.nn.log_softmax(node_logits)[node_sel]
    al = action_logits[node_sel, :]
    act_sel = jax.random.categorical(jax.random.key(43), al)
    act_lp = jax.nn.log_softmax(al)[act_sel]
    return (node_sel, act_sel, node_lp + act_lp)


# speedup vs baseline: 8.7165x; 6.9768x over previous
import functools

import jax
import jax.numpy as jnp
from jax import lax
from jax.experimental import pallas as pl
from jax.experimental.pallas import tpu as pltpu
from jax.experimental.pallas import tpu_sc as plsc

NNODE = 10000
NPAD = 10240
NEDGE = 320000
NEG = 0.2
NW = 32
EPW = NEDGE // NW
CB = 40
NCH = EPW // CB


def _lr(x):
    return jnp.where(x >= 0, x, NEG * x)


def _allsum16(v):
    io = lax.iota(jnp.int32, 16)
    dn = lax.GatherDimensionNumbers(
        offset_dims=(), collapsed_slice_dims=(0,), start_index_map=(0,))
    for sh in (8, 4, 2, 1):
        p = lax.gather(v, (io ^ sh)[:, None], dn, (1,),
                       mode=lax.GatherScatterMode.PROMISE_IN_BOUNDS)
        v = v + p
    return v


def _mm_body(x_ref, w_ref, o_ref):
    o_ref[...] = jnp.dot(x_ref[...], w_ref[...], preferred_element_type=jnp.float32)


def _matmul(x, w, blk_rows):
    n = x.shape[0]
    return pl.pallas_call(
        _mm_body,
        grid=(n // blk_rows,),
        in_specs=[
            pl.BlockSpec((blk_rows, x.shape[1]), lambda i: (i, 0)),
            pl.BlockSpec(w.shape, lambda i: (0, 0)),
        ],
        out_specs=pl.BlockSpec((blk_rows, w.shape[1]), lambda i: (i, 0)),
        out_shape=jax.ShapeDtypeStruct((n, w.shape[1]), jnp.float32),
    )(x, w)


def _edge_proj_body(ea_ref, w1_ref, w2_ref, o1_ref, o2_ref):
    a = ea_ref[...]
    o1_ref[...] = jnp.dot(a, w1_ref[...], preferred_element_type=jnp.float32)
    o2_ref[...] = jnp.dot(a, w2_ref[...], preferred_element_type=jnp.float32)


def _edge_proj(ea_r, wbd1, wbd23):
    n = ea_r.shape[0]
    blk = 1600
    return pl.pallas_call(
        _edge_proj_body,
        grid=(n // blk,),
        in_specs=[
            pl.BlockSpec((blk, 128), lambda i: (i, 0)),
            pl.BlockSpec((128, 1024), lambda i: (0, 0)),
            pl.BlockSpec((128, 256), lambda i: (0, 0)),
        ],
        out_specs=[
            pl.BlockSpec((blk, 1024), lambda i: (i, 0)),
            pl.BlockSpec((blk, 256), lambda i: (i, 0)),
        ],
        out_shape=[
            jax.ShapeDtypeStruct((n, 1024), jnp.float32),
            jax.ShapeDtypeStruct((n, 256), jnp.float32),
        ],
    )(ea_r, wbd1, wbd23)


def _combine1_body(p0_ref, p1_ref, xl_ref, we1_ref, att1_ref, b1_ref,
                   w23_ref, we23_ref, att23_ref, xf_ref, es_ref):
    s = p0_ref[...] + p1_ref[...]
    accv = s[:, :128]
    eesum = s[:, 128:129]
    cnt = s[:, 129:130]
    easum = s[:, 144:160]
    mean = easum / jnp.maximum(cnt, 1.0)
    xl = xl_ref[...]
    mw1 = jnp.dot(mean, we1_ref[...], preferred_element_type=jnp.float32)
    t1 = _lr(2.0 * xl + mw1)
    e1s = jnp.sum(t1 * att1_ref[...], axis=1, keepdims=True)
    ee1 = jnp.exp(e1s)
    latent = (accv + ee1 * xl) / (eesum + ee1) + b1_ref[...]
    xf = jnp.dot(latent, w23_ref[...], preferred_element_type=jnp.float32)
    mw23 = jnp.dot(mean, we23_ref[...], preferred_element_type=jnp.float32)
    t = _lr(2.0 * xf + mw23)
    w = att23_ref[...]
    e3s = jnp.sum(t[:, :16] * w[:, :16], axis=1, keepdims=True)
    e2s = jnp.sum(t[:, 16:] * w[:, 16:], axis=1, keepdims=True)
    xf_ref[...] = xf
    es_ref[...] = jnp.concatenate(
        [jnp.exp(e2s), jnp.exp(e3s), jnp.zeros((xf.shape[0], 6), jnp.float32)], axis=1)


def _combine1(p0, p1, xl1, we1, att1, b1, w23, we23, att23):
    blk = 400
    return pl.pallas_call(
        _combine1_body,
        grid=(NNODE // blk,),
        in_specs=[
            pl.BlockSpec((blk, 160), lambda i: (i, 0)),
            pl.BlockSpec((blk, 160), lambda i: (i, 0)),
            pl.BlockSpec((blk, 128), lambda i: (i, 0)),
            pl.BlockSpec((16, 128), lambda i: (0, 0)),
            pl.BlockSpec((1, 128), lambda i: (0, 0)),
            pl.BlockSpec((1, 128), lambda i: (0, 0)),
            pl.BlockSpec((128, 32), lambda i: (0, 0)),
            pl.BlockSpec((16, 32), lambda i: (0, 0)),
            pl.BlockSpec((1, 32), lambda i: (0, 0)),
        ],
        out_specs=[
            pl.BlockSpec((blk, 32), lambda i: (i, 0)),
            pl.BlockSpec((blk, 8), lambda i: (i, 0)),
        ],
        out_shape=[
            jax.ShapeDtypeStruct((NNODE, 32), jnp.float32),
            jax.ShapeDtypeStruct((NNODE, 8), jnp.float32),
        ],
    )(p0, p1, xl1, we1, att1, b1, w23, we23, att23)


def _combine2_body(q0_ref, q1_ref, xf_ref, es_ref, b2_ref, b3_ref, nl_ref, al_ref):
    s = q0_ref[...] + q1_ref[...]
    acc3 = s[:, :16]
    acc2 = s[:, 16:17]
    d2 = s[:, 17:18]
    d3 = s[:, 18:19]
    es = es_ref[...]
    ee2 = es[:, 0:1]
    ee3 = es[:, 1:2]
    xf = xf_ref[...]
    xl2 = xf[:, 16:17]
    xl3 = xf[:, :16]
    nl_ref[...] = (acc2 + ee2 * xl2) / (d2 + ee2) + b2_ref[...]
    al_ref[...] = (acc3 + ee3 * xl3) / (d3 + ee3) + b3_ref[...]


def _combine2(q0, q1, xf, es, b2, b3):
    blk = 400
    return pl.pallas_call(
        _combine2_body,
        grid=(NNODE // blk,),
        in_specs=[
            pl.BlockSpec((blk, 32), lambda i: (i, 0)),
            pl.BlockSpec((blk, 32), lambda i: (i, 0)),
            pl.BlockSpec((blk, 32), lambda i: (i, 0)),
            pl.BlockSpec((blk, 8), lambda i: (i, 0)),
            pl.BlockSpec((1, 1), lambda i: (0, 0)),
            pl.BlockSpec((1, 16), lambda i: (0, 0)),
        ],
        out_specs=[
            pl.BlockSpec((blk, 1), lambda i: (i, 0)),
            pl.BlockSpec((blk, 16), lambda i: (i, 0)),
        ],
        out_shape=[
            jax.ShapeDtypeStruct((NNODE, 1), jnp.float32),
            jax.ShapeDtypeStruct((NNODE, 16), jnp.float32),
        ],
    )(q0, q1, xf, es, b2, b3)


def _sc_mesh():
    return plsc.VectorSubcoreMesh(
        core_axis_name="c", subcore_axis_name="s", num_cores=2, num_subcores=16)


def _sc1_body(xl1_hbm, g1_hbm, src_hbm, dst_hbm, ea_hbm, att1_hbm, z_hbm,
              out_hbm,
              acc_sh, srcv, dstv, g1v, eav, xsv, xdv, stage, att1v, sem):
    cid = lax.axis_index("c")
    sid = lax.axis_index("s")
    wid = cid * 16 + sid

    pltpu.sync_copy(z_hbm.at[pl.ds(sid * 640, 640), :],
                    acc_sh.at[pl.ds(sid * 640, 640), :])
    pltpu.sync_copy(att1_hbm, att1v)
    plsc.subcore_barrier()

    attc = [att1v[pl.ds(16 * k, 16)] for k in range(8)]
    io = lax.iota(jnp.int32, 16)
    l0 = jnp.where(io == 0, 1.0, 0.0).astype(jnp.float32)
    l1 = jnp.where(io == 1, 1.0, 0.0).astype(jnp.float32)
    ebase = wid * EPW

    @pl.loop(0, NCH)
    def _chunk(ch):
        base = ebase + ch * CB
        pltpu.sync_copy(src_hbm.at[pl.ds(base, CB)], srcv)
        pltpu.sync_copy(dst_hbm.at[pl.ds(base, CB)], dstv)
        pltpu.sync_copy(g1_hbm.at[pl.ds(base, CB), :], g1v)
        pltpu.sync_copy(ea_hbm.at[pl.ds(base, CB), :], eav)
        pltpu.async_copy(xl1_hbm.at[srcv], xsv, sem).wait()
        pltpu.async_copy(xl1_hbm.at[dstv], xdv, sem).wait()

        @pl.loop(0, CB)
        def _edge(i):
            xs = []
            acc = jnp.zeros((16,), jnp.float32)
            for k in range(8):
                a = xsv[i, pl.ds(16 * k, 16)]
                m = a + xdv[i, pl.ds(16 * k, 16)] + g1v[i, pl.ds(16 * k, 16)]
                acc = acc + _lr(m) * attc[k]
                xs.append(a)
            ee = jnp.exp(_allsum16(acc))
            for k in range(8):
                stage[i, pl.ds(16 * k, 16)] = xs[k] * ee
            stage[i, pl.ds(128, 16)] = ee * l0 + l1
            stage[i, pl.ds(144, 16)] = eav[i, :]

        pltpu.sync_copy(stage, acc_sh.at[dstv], add=True)

    plsc.subcore_barrier()
    pltpu.sync_copy(acc_sh.at[pl.ds(sid * 640, 640), :],
                    out_hbm.at[cid, pl.ds(sid * 640, 640), :])


def _sc1_call(xl1, g1, src, dst, ea, att1, z160):
    f = pl.kernel(
        _sc1_body,
        out_type=jax.ShapeDtypeStruct((2, NPAD, 160), jnp.float32),
        mesh=_sc_mesh(),
        compiler_params=pltpu.CompilerParams(use_tc_tiling_on_sc=False),
        scratch_types=[
            pltpu.VMEM_SHARED((NPAD, 160), jnp.float32),
            pltpu.VMEM((CB,), jnp.int32),
            pltpu.VMEM((CB,), jnp.int32),
            pltpu.VMEM((CB, 128), jnp.float32),
            pltpu.VMEM((CB, 16), jnp.float32),
            pltpu.VMEM((CB, 128), jnp.float32),
            pltpu.VMEM((CB, 128), jnp.float32),
            pltpu.VMEM((CB, 160), jnp.float32),
            pltpu.VMEM((128,), jnp.float32),
            pltpu.SemaphoreType.DMA,
        ],
    )
    return f(xl1, g1, src, dst, ea, att1, z160)


def _sc23_body(xf_hbm, g23_hbm, src_hbm, dst_hbm, att23_hbm, z_hbm,
               out_hbm,
               acc_sh, srcv, dstv, g23v, xsv, xdv, stage, att23v, sem):
    cid = lax.axis_index("c")
    sid = lax.axis_index("s")
    wid = cid * 16 + sid

    pltpu.sync_copy(z_hbm.at[pl.ds(sid * 640, 640), :],
                    acc_sh.at[pl.ds(sid * 640, 640), :])
    pltpu.sync_copy(att23_hbm, att23v)
    plsc.subcore_barrier()

    att3 = att23v[pl.ds(0, 16)]
    att2h = att23v[pl.ds(16, 16)]
    io = lax.iota(jnp.int32, 16)
    l0 = jnp.where(io == 0, 1.0, 0.0).astype(jnp.float32)
    l1 = jnp.where(io == 1, 1.0, 0.0).astype(jnp.float32)
    l2 = jnp.where(io == 2, 1.0, 0.0).astype(jnp.float32)
    ebase = wid * EPW

    @pl.loop(0, NCH)
    def _chunk(ch):
        base = ebase + ch * CB
        pltpu.sync_copy(src_hbm.at[pl.ds(base, CB)], srcv)
        pltpu.sync_copy(dst_hbm.at[pl.ds(base, CB)], dstv)
        pltpu.sync_copy(g23_hbm.at[pl.ds(base, CB), :], g23v)
        pltpu.async_copy(xf_hbm.at[srcv], xsv, sem).wait()
        pltpu.async_copy(xf_hbm.at[dstv], xdv, sem).wait()

        @pl.loop(0, CB)
        def _edge(i):
            xs_lo = xsv[i, pl.ds(0, 16)]
            xs_hi = xsv[i, pl.ds(16, 16)]
            m3 = xs_lo + xdv[i, pl.ds(0, 16)] + g23v[i, pl.ds(0, 16)]
            v2 = xs_hi + xdv[i, pl.ds(16, 16)] + g23v[i, pl.ds(16, 16)]
            ee3 = jnp.exp(_allsum16(_lr(m3) * att3))
            ee2 = jnp.exp(_allsum16(_lr(v2) * att2h))
            stage[i, pl.ds(0, 16)] = xs_lo * ee3
            stage[i, pl.ds(16, 16)] = ee2 * (xs_hi * l0 + l1) + ee3 * l2

        pltpu.sync_copy(stage, acc_sh.at[dstv], add=True)

    plsc.subcore_barrier()
    pltpu.sync_copy(acc_sh.at[pl.ds(sid * 640, 640), :],
                    out_hbm.at[cid, pl.ds(sid * 640, 640), :])


def _sc23_call(xf, g23, src, dst, att23, z32):
    f = pl.kernel(
        _sc23_body,
        out_type=jax.ShapeDtypeStruct((2, NPAD, 32), jnp.float32),
        mesh=_sc_mesh(),
        compiler_params=pltpu.CompilerParams(use_tc_tiling_on_sc=False),
        scratch_types=[
            pltpu.VMEM_SHARED((NPAD, 32), jnp.float32),
            pltpu.VMEM((CB,), jnp.int32),
            pltpu.VMEM((CB,), jnp.int32),
            pltpu.VMEM((CB, 32), jnp.float32),
            pltpu.VMEM((CB, 32), jnp.float32),
            pltpu.VMEM((CB, 32), jnp.float32),
            pltpu.VMEM((CB, 32), jnp.float32),
            pltpu.VMEM((32,), jnp.float32),
            pltpu.SemaphoreType.DMA,
        ],
    )
    return f(xf, g23, src, dst, att23, z32)


def kernel(x, edge_index, edge_attr, W1, att1, We1, b1, W2, att2, We2, b2,
           W3, att3, We3, b3):
    f32 = jnp.float32
    src = edge_index[0]
    dst = edge_index[1]

    ea_r = edge_attr.reshape(NEDGE // 8, 128)
    we23 = jnp.concatenate(
        [We3, We2, jnp.zeros((16, 15), f32)], axis=1)
    z1 = jnp.zeros((128, 1024), f32)
    z23 = jnp.zeros((128, 256), f32)
    idx8 = jnp.arange(8)
    wbd1 = z1.reshape(8, 16, 8, 128).at[idx8, :, idx8, :].set(We1).reshape(128, 1024)
    wbd23 = z23.reshape(8, 16, 8, 32).at[idx8, :, idx8, :].set(we23).reshape(128, 256)

    xl1 = _matmul(x, W1, 400)
    g1f, g23f = _edge_proj(ea_r, wbd1, wbd23)
    g1 = g1f.reshape(NEDGE, 128)
    g23 = g23f.reshape(NEDGE, 32)

    z160 = jnp.zeros((NPAD, 160), f32)
    p = _sc1_call(xl1, g1, src, dst, edge_attr, att1, z160)
    p0 = p[0, :NNODE, :]
    p1 = p[1, :NNODE, :]

    att23 = jnp.concatenate([att3, att2, jnp.zeros((15,), f32)]).reshape(1, 32)
    w23 = jnp.concatenate([W3, W2, jnp.zeros((128, 15), f32)], axis=1)
    xf, es = _combine1(p0, p1, xl1, We1, att1.reshape(1, 128),
                       b1.reshape(1, 128), w23, we23, att23)

    z32 = jnp.zeros((NPAD, 32), f32)
    q = _sc23_call(xf, g23, src, dst, att23.reshape(32), z32)
    q0 = q[0, :NNODE, :]
    q1 = q[1, :NNODE, :]

    nl, al = _combine2(q0, q1, xf, es, b2.reshape(1, 1), b3.reshape(1, 16))
    node_logits = nl[:, 0]
    action_logits = al

    node_sel = jax.random.categorical(jax.random.key(42), node_logits)
    node_lp = jax.nn.log_softmax(node_logits)[node_sel]
    alr = action_logits[node_sel, :]
    act_sel = jax.random.categorical(jax.random.key(43), alr)
    act_lp = jax.nn.log_softmax(alr)[act_sel]
    return (node_sel, act_sel, node_lp + act_lp)


# speedup vs baseline: 17.2050x; 1.9738x over previous
import jax
import jax.numpy as jnp
from jax import lax
from jax.experimental import pallas as pl
from jax.experimental.pallas import tpu as pltpu
from jax.experimental.pallas import tpu_sc as plsc

NNODE = 10000
NEDGE = 320000
NEG = 0.2
NW = 32
EPW = NEDGE // NW
RPT = NNODE // 16
CB1 = 16
NCH1 = EPW // CB1
CB2 = 80
NCH2 = EPW // CB2


def _lr(x):
    return jnp.where(x >= 0, x, NEG * x)


def _allsum16(v):
    io = lax.iota(jnp.int32, 16)
    dn = lax.GatherDimensionNumbers(
        offset_dims=(), collapsed_slice_dims=(0,), start_index_map=(0,))
    for sh in (8, 4, 2, 1):
        p = lax.gather(v, (io ^ sh)[:, None], dn, (1,),
                       mode=lax.GatherScatterMode.PROMISE_IN_BOUNDS)
        v = v + p
    return v


def _mm_body(x_ref, w_ref, o_ref):
    o_ref[...] = jnp.dot(x_ref[...], w_ref[...], preferred_element_type=jnp.float32)


def _matmul(x, w, blk_rows):
    n = x.shape[0]
    return pl.pallas_call(
        _mm_body,
        grid=(n // blk_rows,),
        in_specs=[
            pl.BlockSpec((blk_rows, x.shape[1]), lambda i: (i, 0)),
            pl.BlockSpec(w.shape, lambda i: (0, 0)),
        ],
        out_specs=pl.BlockSpec((blk_rows, w.shape[1]), lambda i: (i, 0)),
        out_shape=jax.ShapeDtypeStruct((n, w.shape[1]), jnp.float32),
    )(x, w)


def _edge_proj_body(ea_ref, w1_ref, w2_ref, o1_ref, o2_ref):
    a = ea_ref[...]
    o1_ref[...] = jnp.dot(a, w1_ref[...], preferred_element_type=jnp.float32)
    o2_ref[...] = jnp.dot(a, w2_ref[...], preferred_element_type=jnp.float32)


def _edge_proj(ea_r, wbd1, wbd23):
    n = ea_r.shape[0]
    blk = 1600
    return pl.pallas_call(
        _edge_proj_body,
        grid=(n // blk,),
        in_specs=[
            pl.BlockSpec((blk, 128), lambda i: (i, 0)),
            pl.BlockSpec((128, 1024), lambda i: (0, 0)),
            pl.BlockSpec((128, 256), lambda i: (0, 0)),
        ],
        out_specs=[
            pl.BlockSpec((blk, 1024), lambda i: (i, 0)),
            pl.BlockSpec((blk, 256), lambda i: (i, 0)),
        ],
        out_shape=[
            jax.ShapeDtypeStruct((n, 1024), jnp.float32),
            jax.ShapeDtypeStruct((n, 256), jnp.float32),
        ],
    )(ea_r, wbd1, wbd23)


def _combine1_body(p0_ref, p1_ref, e0_ref, e1_ref, m0_ref, m1_ref, c0_ref, c1_ref,
                   xl_ref, we1_ref, att1_ref, b1_ref,
                   w23_ref, we23_ref, att23_ref, xf_ref, es_ref):
    accv = p0_ref[...] + p1_ref[...]
    eesum = e0_ref[...][:, 0:1] + e1_ref[...][:, 0:1]
    cnt = c0_ref[...][:, 0:1] + c1_ref[...][:, 0:1]
    easum = m0_ref[...] + m1_ref[...]
    mean = easum / jnp.maximum(cnt, 1.0)
    xl = xl_ref[...]
    mw1 = jnp.dot(mean, we1_ref[...], preferred_element_type=jnp.float32)
    t1 = _lr(2.0 * xl + mw1)
    e1s = jnp.sum(t1 * att1_ref[...], axis=1, keepdims=True)
    ee1 = jnp.exp(e1s)
    latent = (accv + ee1 * xl) / (eesum + ee1) + b1_ref[...]
    xf = jnp.dot(latent, w23_ref[...], preferred_element_type=jnp.float32)
    mw23 = jnp.dot(mean, we23_ref[...], preferred_element_type=jnp.float32)
    t = _lr(2.0 * xf + mw23)
    w = att23_ref[...]
    e3s = jnp.sum(t[:, :16] * w[:, :16], axis=1, keepdims=True)
    e2s = jnp.sum(t[:, 16:] * w[:, 16:], axis=1, keepdims=True)
    xf_ref[...] = xf
    es_ref[...] = jnp.concatenate(
        [jnp.exp(e2s), jnp.exp(e3s), jnp.zeros((xf.shape[0], 6), jnp.float32)], axis=1)


def _combine1(p0, p1, e0, e1, m0, m1, c0, c1, xl1, we1, att1, b1, w23, we23, att23):
    blk = 400
    return pl.pallas_call(
        _combine1_body,
        grid=(NNODE // blk,),
        in_specs=[
            pl.BlockSpec((blk, 128), lambda i: (i, 0)),
            pl.BlockSpec((blk, 128), lambda i: (i, 0)),
            pl.BlockSpec((blk, 16), lambda i: (i, 0)),
            pl.BlockSpec((blk, 16), lambda i: (i, 0)),
            pl.BlockSpec((blk, 16), lambda i: (i, 0)),
            pl.BlockSpec((blk, 16), lambda i: (i, 0)),
            pl.BlockSpec((blk, 16), lambda i: (i, 0)),
            pl.BlockSpec((blk, 16), lambda i: (i, 0)),
            pl.BlockSpec((blk, 128), lambda i: (i, 0)),
            pl.BlockSpec((16, 128), lambda i: (0, 0)),
            pl.BlockSpec((1, 128), lambda i: (0, 0)),
            pl.BlockSpec((1, 128), lambda i: (0, 0)),
            pl.BlockSpec((128, 32), lambda i: (0, 0)),
            pl.BlockSpec((16, 32), lambda i: (0, 0)),
            pl.BlockSpec((1, 32), lambda i: (0, 0)),
        ],
        out_specs=[
            pl.BlockSpec((blk, 32), lambda i: (i, 0)),
            pl.BlockSpec((blk, 8), lambda i: (i, 0)),
        ],
        out_shape=[
            jax.ShapeDtypeStruct((NNODE, 32), jnp.float32),
            jax.ShapeDtypeStruct((NNODE, 8), jnp.float32),
        ],
    )(p0, p1, e0, e1, m0, m1, c0, c1, xl1, we1, att1, b1, w23, we23, att23)


def _combine2_body(q0_ref, q1_ref, xf_ref, es_ref, b2_ref, b3_ref, nl_ref, al_ref):
    s = q0_ref[...] + q1_ref[...]
    acc3 = s[:, :16]
    acc2 = s[:, 16:17]
    d2 = s[:, 17:18]
    d3 = s[:, 18:19]
    es = es_ref[...]
    ee2 = es[:, 0:1]
    ee3 = es[:, 1:2]
    xf = xf_ref[...]
    nl_ref[...] = (acc2 + ee2 * xf[:, 16:17]) / (d2 + ee2) + b2_ref[...]
    al_ref[...] = (acc3 + ee3 * xf[:, :16]) / (d3 + ee3) + b3_ref[...]


def _combine2(q0, q1, xf, es, b2, b3):
    blk = 400
    return pl.pallas_call(
        _combine2_body,
        grid=(NNODE // blk,),
        in_specs=[
            pl.BlockSpec((blk, 32), lambda i: (i, 0)),
            pl.BlockSpec((blk, 32), lambda i: (i, 0)),
            pl.BlockSpec((blk, 32), lambda i: (i, 0)),
            pl.BlockSpec((blk, 8), lambda i: (i, 0)),
            pl.BlockSpec((1, 1), lambda i: (0, 0)),
            pl.BlockSpec((1, 16), lambda i: (0, 0)),
        ],
        out_specs=[
            pl.BlockSpec((blk, 1), lambda i: (i, 0)),
            pl.BlockSpec((blk, 16), lambda i: (i, 0)),
        ],
        out_shape=[
            jax.ShapeDtypeStruct((NNODE, 1), jnp.float32),
            jax.ShapeDtypeStruct((NNODE, 16), jnp.float32),
        ],
    )(q0, q1, xf, es, b2, b3)


def _sc_mesh():
    return plsc.VectorSubcoreMesh(
        core_axis_name="c", subcore_axis_name="s", num_cores=2, num_subcores=16)


def _sc0_body(ei_hbm, ea_hbm, z_hbm, mout_hbm, cout_hbm,
              accm, accc, dstv, eav, ones, sem_d, sem_e, sem_sm, sem_sc):
    cid = lax.axis_index("c")
    sid = lax.axis_index("s")
    ebase = _wid_of(cid, sid) * EPW
    r0 = sid * RPT

    pltpu.sync_copy(z_hbm.at[pl.ds(r0, RPT), pl.ds(0, 16)],
                    accm.at[pl.ds(r0, RPT), :])
    pltpu.sync_copy(z_hbm.at[pl.ds(r0, RPT), pl.ds(16, 16)],
                    accc.at[pl.ds(r0, RPT), :])

    @pl.loop(0, CB2)
    def _init(i):
        ones[i, :] = jnp.where(lax.iota(jnp.int32, 16) == 0, 1.0, 0.0)

    plsc.subcore_barrier()

    def d_dst(ch, s):
        return pltpu.make_async_copy(
            ei_hbm.at[1, pl.ds(ebase + ch * CB2, CB2)], dstv.at[s], sem_d.at[s])

    def d_ea(ch, s):
        return pltpu.make_async_copy(
            ea_hbm.at[pl.ds(ebase + ch * CB2, CB2), :], eav.at[s], sem_e.at[s])

    def d_sm(s, b):
        return pltpu.make_async_copy(eav.at[s], accm.at[dstv.at[s]], sem_sm.at[b])

    def d_sc(s, b):
        return pltpu.make_async_copy(ones, accc.at[dstv.at[s]], sem_sc.at[b])

    pltpu.sync_copy(ei_hbm.at[1, pl.ds(ebase, CB2)], dstv.at[0])
    pltpu.sync_copy(ea_hbm.at[pl.ds(ebase, CB2), :], eav.at[0])
    d_dst(1, 1).start()
    d_ea(1, 1).start()

    @pl.loop(0, NCH2)
    def _chunk(ch):
        s4 = lax.rem(ch, 4)
        for ss in range(4):
            @pl.when(s4 == ss)
            def _():
                bb = ss & 1
                s1 = (ss + 1) % 4
                s2 = (ss + 2) % 4

                @pl.when(ch + 1 < NCH2)
                def _():
                    d_dst(ch + 1, s1).wait()
                    d_ea(ch + 1, s1).wait()

                @pl.when(ch >= 2)
                def _():
                    d_sm(s2, bb).wait()
                    d_sc(s2, bb).wait()

                @pl.when(ch + 2 < NCH2)
                def _():
                    d_dst(ch + 2, s2).start()
                    d_ea(ch + 2, s2).start()

                d_sm(ss, bb).start(add=True)
                d_sc(ss, bb).start(add=True)

    for chl in (NCH2 - 2, NCH2 - 1):
        d_sm(chl % 4, chl % 2).wait()
        d_sc(chl % 4, chl % 2).wait()

    plsc.subcore_barrier()
    pltpu.sync_copy(accm.at[pl.ds(r0, RPT), :], mout_hbm.at[cid, pl.ds(r0, RPT), :])
    pltpu.sync_copy(accc.at[pl.ds(r0, RPT), :], cout_hbm.at[cid, pl.ds(r0, RPT), :])


def _wid_of(cid, sid):
    return cid * 16 + sid


def _sc0_call(edge_index, edge_attr, z32):
    f = pl.kernel(
        _sc0_body,
        out_type=(jax.ShapeDtypeStruct((2, NNODE, 16), jnp.float32),
                  jax.ShapeDtypeStruct((2, NNODE, 16), jnp.float32)),
        mesh=_sc_mesh(),
        compiler_params=pltpu.CompilerParams(use_tc_tiling_on_sc=False),
        scratch_types=[
            pltpu.VMEM_SHARED((NNODE, 16), jnp.float32),
            pltpu.VMEM_SHARED((NNODE, 16), jnp.float32),
            pltpu.VMEM((4, CB2), jnp.int32),
            pltpu.VMEM((4, CB2, 16), jnp.float32),
            pltpu.VMEM((CB2, 16), jnp.float32),
            pltpu.SemaphoreType.DMA((4,)),
            pltpu.SemaphoreType.DMA((4,)),
            pltpu.SemaphoreType.DMA((2,)),
            pltpu.SemaphoreType.DMA((2,)),
        ],
    )
    return f(edge_index, edge_attr, z32)


def _sc1_body(xl1_hbm, g1_hbm, ei_hbm, att1_hbm, z128_hbm, z16_hbm,
              pout_hbm, eout_hbm,
              accp, acce, sd, comb, xdv, g1v, meta, att1v,
              sem_sd, sem_xs, sem_xd, sem_g, sem_scm, sem_sce):
    cid = lax.axis_index("c")
    sid = lax.axis_index("s")
    ebase = _wid_of(cid, sid) * EPW
    r0 = sid * RPT

    pltpu.sync_copy(z128_hbm.at[pl.ds(r0, RPT), :], accp.at[pl.ds(r0, RPT), :])
    pltpu.sync_copy(z16_hbm.at[pl.ds(r0, RPT), :], acce.at[pl.ds(r0, RPT), :])
    pltpu.sync_copy(att1_hbm, att1v)
    plsc.subcore_barrier()

    attc = [att1v[pl.ds(16 * k, 16)] for k in range(8)]
    io = lax.iota(jnp.int32, 16)
    l0 = jnp.where(io == 0, 1.0, 0.0).astype(jnp.float32)

    def d_sd(ch, s):
        return pltpu.make_async_copy(
            ei_hbm.at[:, pl.ds(ebase + ch * CB1, CB1)], sd.at[s], sem_sd.at[s])

    def d_xs(ch, s):
        return pltpu.make_async_copy(
            xl1_hbm.at[sd.at[s].at[0]], comb.at[s], sem_xs.at[s & 1])

    def d_xd(ch, s):
        return pltpu.make_async_copy(
            xl1_hbm.at[sd.at[s].at[1]], xdv.at[s & 1], sem_xd.at[s & 1])

    def d_g(ch, s):
        return pltpu.make_async_copy(
            g1_hbm.at[pl.ds(ebase + ch * CB1, CB1), :], g1v.at[s & 1], sem_g.at[s & 1])

    def d_scm(s):
        return pltpu.make_async_copy(
            comb.at[s], accp.at[sd.at[s].at[1]], sem_scm.at[s & 1])

    def d_sce(s):
        return pltpu.make_async_copy(
            meta.at[s & 1], acce.at[sd.at[s].at[1]], sem_sce.at[s & 1])

    pltpu.sync_copy(ei_hbm.at[:, pl.ds(ebase, CB1)], sd.at[0])
    d_xs(0, 0).start()
    d_xd(0, 0).start()
    d_g(0, 0).start()
    d_sd(1, 1).start()

    @pl.loop(0, NCH1)
    def _chunk(ch):
        s4 = lax.rem(ch, 4)
        for ss in range(4):
            @pl.when(s4 == ss)
            def _():
                bb = ss & 1
                s1 = (ss + 1) % 4
                s2 = (ss + 2) % 4

                @pl.when(ch + 1 < NCH1)
                def _():
                    d_sd(ch + 1, s1).wait()

                @pl.when(ch >= 2)
                def _():
                    d_scm(s2).wait()
                    d_sce(s2).wait()

                @pl.when(ch + 1 < NCH1)
                def _():
                    d_xs(ch + 1, s1).start()
                    d_xd(ch + 1, s1).start()
                    d_g(ch + 1, s1).start()

                @pl.when(ch + 2 < NCH1)
                def _():
                    d_sd(ch + 2, s2).start()

                d_xs(ch, ss).wait()
                d_xd(ch, ss).wait()
                d_g(ch, ss).wait()

                @pl.loop(0, CB1)
                def _edge(i):
                    xs = []
                    acc = jnp.zeros((16,), jnp.float32)
                    for k in range(8):
                        a = comb[ss, i, pl.ds(16 * k, 16)]
                        m = (a + xdv[bb, i, pl.ds(16 * k, 16)]
                             + g1v[bb, i, pl.ds(16 * k, 16)])
                        acc = acc + _lr(m) * attc[k]
                        xs.append(a)
                    ee = jnp.exp(_allsum16(acc))
                    for k in range(8):
                        comb[ss, i, pl.ds(16 * k, 16)] = xs[k] * ee
                    meta[bb, i, :] = ee * l0

                d_scm(ss).start(add=True)
                d_sce(ss).start(add=True)

    for chl in (NCH1 - 2, NCH1 - 1):
        d_scm(chl % 4).wait()
        d_sce(chl % 4).wait()

    plsc.subcore_barrier()
    pltpu.sync_copy(accp.at[pl.ds(r0, RPT), :], pout_hbm.at[cid, pl.ds(r0, RPT), :])
    pltpu.sync_copy(acce.at[pl.ds(r0, RPT), :], eout_hbm.at[cid, pl.ds(r0, RPT), :])


def _sc1_call(xl1, g1, edge_index, att1, z128, z16):
    f = pl.kernel(
        _sc1_body,
        out_type=(jax.ShapeDtypeStruct((2, NNODE, 128), jnp.float32),
                  jax.ShapeDtypeStruct((2, NNODE, 16), jnp.float32)),
        mesh=_sc_mesh(),
        compiler_params=pltpu.CompilerParams(use_tc_tiling_on_sc=False),
        scratch_types=[
            pltpu.VMEM_SHARED((NNODE, 128), jnp.float32),
            pltpu.VMEM_SHARED((NNODE, 16), jnp.float32),
            pltpu.VMEM((4, 2, CB1), jnp.int32),
            pltpu.VMEM((4, CB1, 128), jnp.float32),
            pltpu.VMEM((2, CB1, 128), jnp.float32),
            pltpu.VMEM((2, CB1, 128), jnp.float32),
            pltpu.VMEM((2, CB1, 16), jnp.float32),
            pltpu.VMEM((128,), jnp.float32),
            pltpu.SemaphoreType.DMA((4,)),
            pltpu.SemaphoreType.DMA((2,)),
            pltpu.SemaphoreType.DMA((2,)),
            pltpu.SemaphoreType.DMA((2,)),
            pltpu.SemaphoreType.DMA((2,)),
            pltpu.SemaphoreType.DMA((2,)),
        ],
    )
    return f(xl1, g1, edge_index, att1, z128, z16)


def _sc23_body(xf_hbm, g23_hbm, ei_hbm, att23_hbm, z32_hbm, out_hbm,
               acc_sh, sd, comb, xdv, g23v, att23v,
               sem_sd, sem_xs, sem_xd, sem_g, sem_sc):
    cid = lax.axis_index("c")
    sid = lax.axis_index("s")
    ebase = _wid_of(cid, sid) * EPW
    r0 = sid * RPT

    pltpu.sync_copy(z32_hbm.at[pl.ds(r0, RPT), :], acc_sh.at[pl.ds(r0, RPT), :])
    pltpu.sync_copy(att23_hbm, att23v)
    plsc.subcore_barrier()

    att3 = att23v[pl.ds(0, 16)]
    att2h = att23v[pl.ds(16, 16)]
    io = lax.iota(jnp.int32, 16)
    l0 = jnp.where(io == 0, 1.0, 0.0).astype(jnp.float32)
    l1 = jnp.where(io == 1, 1.0, 0.0).astype(jnp.float32)
    l2 = jnp.where(io == 2, 1.0, 0.0).astype(jnp.float32)

    def d_sd(ch, s):
        return pltpu.make_async_copy(
            ei_hbm.at[:, pl.ds(ebase + ch * CB2, CB2)], sd.at[s], sem_sd.at[s])

    def d_xs(ch, s):
        return pltpu.make_async_copy(
            xf_hbm.at[sd.at[s].at[0]], comb.at[s], sem_xs.at[s & 1])

    def d_xd(ch, s):
        return pltpu.make_async_copy(
            xf_hbm.at[sd.at[s].at[1]], xdv.at[s & 1], sem_xd.at[s & 1])

    def d_g(ch, s):
        return pltpu.make_async_copy(
            g23_hbm.at[pl.ds(ebase + ch * CB2, CB2), :], g23v.at[s & 1],
            sem_g.at[s & 1])

    def d_sc(s):
        return pltpu.make_async_copy(
            comb.at[s], acc_sh.at[sd.at[s].at[1]], sem_sc.at[s & 1])

    pltpu.sync_copy(ei_hbm.at[:, pl.ds(ebase, CB2)], sd.at[0])
    d_xs(0, 0).start()
    d_xd(0, 0).start()
    d_g(0, 0).start()
    d_sd(1, 1).start()

    @pl.loop(0, NCH2)
    def _chunk(ch):
        s4 = lax.rem(ch, 4)
        for ss in range(4):
            @pl.when(s4 == ss)
            def _():
                bb = ss & 1
                s1 = (ss + 1) % 4
                s2 = (ss + 2) % 4

                @pl.when(ch + 1 < NCH2)
                def _():
                    d_sd(ch + 1, s1).wait()

                @pl.when(ch >= 2)
                def _():
                    d_sc(s2).wait()

                @pl.when(ch + 1 < NCH2)
                def _():
                    d_xs(ch + 1, s1).start()
                    d_xd(ch + 1, s1).start()
                    d_g(ch + 1, s1).start()

                @pl.when(ch + 2 < NCH2)
                def _():
                    d_sd(ch + 2, s2).start()

                d_xs(ch, ss).wait()
                d_xd(ch, ss).wait()
                d_g(ch, ss).wait()

                @pl.loop(0, CB2)
                def _edge(i):
                    xs_lo = comb[ss, i, pl.ds(0, 16)]
                    xs_hi = comb[ss, i, pl.ds(16, 16)]
                    m3 = (xs_lo + xdv[bb, i, pl.ds(0, 16)]
                          + g23v[bb, i, pl.ds(0, 16)])
                    v2 = (xs_hi + xdv[bb, i, pl.ds(16, 16)]
                          + g23v[bb, i, pl.ds(16, 16)])
                    ee3 = jnp.exp(_allsum16(_lr(m3) * att3))
                    ee2 = jnp.exp(_allsum16(_lr(v2) * att2h))
                    comb[ss, i, pl.ds(0, 16)] = xs_lo * ee3
                    comb[ss, i, pl.ds(16, 16)] = (
                        ee2 * (xs_hi * l0 + l1) + ee3 * l2)

                d_sc(ss).start(add=True)

    for chl in (NCH2 - 2, NCH2 - 1):
        d_sc(chl % 4).wait()

    plsc.subcore_barrier()
    pltpu.sync_copy(acc_sh.at[pl.ds(r0, RPT), :], out_hbm.at[cid, pl.ds(r0, RPT), :])


def _sc23_call(xf, g23, edge_index, att23, z32):
    f = pl.kernel(
        _sc23_body,
        out_type=jax.ShapeDtypeStruct((2, NNODE, 32), jnp.float32),
        mesh=_sc_mesh(),
        compiler_params=pltpu.CompilerParams(use_tc_tiling_on_sc=False),
        scratch_types=[
            pltpu.VMEM_SHARED((NNODE, 32), jnp.float32),
            pltpu.VMEM((4, 2, CB2), jnp.int32),
            pltpu.VMEM((4, CB2, 32), jnp.float32),
            pltpu.VMEM((2, CB2, 32), jnp.float32),
            pltpu.VMEM((2, CB2, 32), jnp.float32),
            pltpu.VMEM((32,), jnp.float32),
            pltpu.SemaphoreType.DMA((4,)),
            pltpu.SemaphoreType.DMA((2,)),
            pltpu.SemaphoreType.DMA((2,)),
            pltpu.SemaphoreType.DMA((2,)),
            pltpu.SemaphoreType.DMA((2,)),
        ],
    )
    return f(xf, g23, edge_index, att23, z32)


def kernel(x, edge_index, edge_attr, W1, att1, We1, b1, W2, att2, We2, b2,
           W3, att3, We3, b3):
    f32 = jnp.float32

    ea_r = edge_attr.reshape(NEDGE // 8, 128)
    we23 = jnp.concatenate([We3, We2, jnp.zeros((16, 15), f32)], axis=1)
    idx8 = jnp.arange(8)
    wbd1 = jnp.zeros((128, 1024), f32).reshape(8, 16, 8, 128)
    wbd1 = wbd1.at[idx8, :, idx8, :].set(We1).reshape(128, 1024)
    wbd23 = jnp.zeros((128, 256), f32).reshape(8, 16, 8, 32)
    wbd23 = wbd23.at[idx8, :, idx8, :].set(we23).reshape(128, 256)

    xl1 = _matmul(x, W1, 400)
    g1f, g23f = _edge_proj(ea_r, wbd1, wbd23)
    g1 = g1f.reshape(NEDGE, 128)
    g23 = g23f.reshape(NEDGE, 32)

    z32 = jnp.zeros((NNODE, 32), f32)
    z128 = jnp.zeros((NNODE, 128), f32)
    z16 = jnp.zeros((NNODE, 16), f32)

    mp, cp = _sc0_call(edge_index, edge_attr, z32)

    p, e = _sc1_call(xl1, g1, edge_index, att1, z128, z16)

    att23 = jnp.concatenate([att3, att2, jnp.zeros((15,), f32)]).reshape(1, 32)
    w23 = jnp.concatenate([W3, W2, jnp.zeros((128, 15), f32)], axis=1)
    xf, es = _combine1(p[0], p[1], e[0], e[1], mp[0], mp[1], cp[0], cp[1],
                       xl1, We1, att1.reshape(1, 128), b1.reshape(1, 128),
                       w23, we23, att23)

    q = _sc23_call(xf, g23, edge_index, att23.reshape(32), z32)

    nl, al = _combine2(q[0], q[1], xf, es, b2.reshape(1, 1), b3.reshape(1, 16))
    node_logits = nl[:, 0]
    action_logits = al

    node_sel = jax.random.categorical(jax.random.key(42), node_logits)
    node_lp = jax.nn.log_softmax(node_logits)[node_sel]
    alr = action_logits[node_sel, :]
    act_sel = jax.random.categorical(jax.random.key(43), alr)
    act_lp = jax.nn.log_softmax(alr)[act_sel]
    return (node_sel, act_sel, node_lp + act_lp)


# speedup vs baseline: 18.3164x; 1.0646x over previous
import jax
import jax.numpy as jnp
from jax import lax
from jax.experimental import pallas as pl
from jax.experimental.pallas import tpu as pltpu
from jax.experimental.pallas import tpu_sc as plsc

NNODE = 10000
NEDGE = 320000
NEG = 0.2
NW = 32
EPW = NEDGE // NW
RPT = NNODE // 16
CB1 = 16
NCH1 = EPW // CB1
CB2 = 80
NCH2 = EPW // CB2


def _lr(x):
    return jnp.where(x >= 0, x, NEG * x)


def _allsum16(v):
    io = lax.iota(jnp.int32, 16)
    dn = lax.GatherDimensionNumbers(
        offset_dims=(), collapsed_slice_dims=(0,), start_index_map=(0,))
    for sh in (8, 4, 2, 1):
        p = lax.gather(v, (io ^ sh)[:, None], dn, (1,),
                       mode=lax.GatherScatterMode.PROMISE_IN_BOUNDS)
        v = v + p
    return v


def _mm_body(x_ref, w_ref, o_ref):
    o_ref[...] = jnp.dot(x_ref[...], w_ref[...], preferred_element_type=jnp.float32)


def _matmul(x, w, blk_rows):
    n = x.shape[0]
    return pl.pallas_call(
        _mm_body,
        grid=(n // blk_rows,),
        in_specs=[
            pl.BlockSpec((blk_rows, x.shape[1]), lambda i: (i, 0)),
            pl.BlockSpec(w.shape, lambda i: (0, 0)),
        ],
        out_specs=pl.BlockSpec((blk_rows, w.shape[1]), lambda i: (i, 0)),
        out_shape=jax.ShapeDtypeStruct((n, w.shape[1]), jnp.float32),
    )(x, w)


def _edge_proj_body(ea_ref, w1_ref, w2_ref, o1_ref, o2_ref):
    a = ea_ref[...]
    o1_ref[...] = jnp.dot(a, w1_ref[...], preferred_element_type=jnp.float32)
    o2_ref[...] = jnp.dot(a, w2_ref[...], preferred_element_type=jnp.float32)


def _edge_proj(ea, we1, we23):
    blk = 4000
    return pl.pallas_call(
        _edge_proj_body,
        grid=(NEDGE // blk,),
        in_specs=[
            pl.BlockSpec((blk, 16), lambda i: (i, 0)),
            pl.BlockSpec((16, 128), lambda i: (0, 0)),
            pl.BlockSpec((16, 32), lambda i: (0, 0)),
        ],
        out_specs=[
            pl.BlockSpec((blk, 128), lambda i: (i, 0)),
            pl.BlockSpec((blk, 32), lambda i: (i, 0)),
        ],
        out_shape=[
            jax.ShapeDtypeStruct((NEDGE, 128), jnp.float32),
            jax.ShapeDtypeStruct((NEDGE, 32), jnp.float32),
        ],
    )(ea, we1, we23)


def _combine1_body(p0_ref, p1_ref, e0_ref, e1_ref, m0_ref, m1_ref, c0_ref, c1_ref,
                   xl_ref, we1_ref, att1_ref, b1_ref,
                   w23_ref, we23_ref, att23_ref, xf_ref, es_ref):
    accv = p0_ref[...] + p1_ref[...]
    eesum = e0_ref[...][:, 0:1] + e1_ref[...][:, 0:1]
    cnt = c0_ref[...][:, 0:1] + c1_ref[...][:, 0:1]
    easum = m0_ref[...] + m1_ref[...]
    mean = easum / jnp.maximum(cnt, 1.0)
    xl = xl_ref[...]
    mw1 = jnp.dot(mean, we1_ref[...], preferred_element_type=jnp.float32)
    t1 = _lr(2.0 * xl + mw1)
    e1s = jnp.sum(t1 * att1_ref[...], axis=1, keepdims=True)
    ee1 = jnp.exp(e1s)
    latent = (accv + ee1 * xl) / (eesum + ee1) + b1_ref[...]
    xf = jnp.dot(latent, w23_ref[...], preferred_element_type=jnp.float32)
    mw23 = jnp.dot(mean, we23_ref[...], preferred_element_type=jnp.float32)
    t = _lr(2.0 * xf + mw23)
    w = att23_ref[...]
    e3s = jnp.sum(t[:, :16] * w[:, :16], axis=1, keepdims=True)
    e2s = jnp.sum(t[:, 16:] * w[:, 16:], axis=1, keepdims=True)
    xf_ref[...] = xf
    es_ref[...] = jnp.concatenate(
        [jnp.exp(e2s), jnp.exp(e3s), jnp.zeros((xf.shape[0], 6), jnp.float32)], axis=1)


def _combine1(p0, p1, e0, e1, m0, m1, c0, c1, xl1, we1, att1, b1, w23, we23, att23):
    blk = 400
    return pl.pallas_call(
        _combine1_body,
        grid=(NNODE // blk,),
        in_specs=[
            pl.BlockSpec((blk, 128), lambda i: (i, 0)),
            pl.BlockSpec((blk, 128), lambda i: (i, 0)),
            pl.BlockSpec((blk, 16), lambda i: (i, 0)),
            pl.BlockSpec((blk, 16), lambda i: (i, 0)),
            pl.BlockSpec((blk, 16), lambda i: (i, 0)),
            pl.BlockSpec((blk, 16), lambda i: (i, 0)),
            pl.BlockSpec((blk, 16), lambda i: (i, 0)),
            pl.BlockSpec((blk, 16), lambda i: (i, 0)),
            pl.BlockSpec((blk, 128), lambda i: (i, 0)),
            pl.BlockSpec((16, 128), lambda i: (0, 0)),
            pl.BlockSpec((1, 128), lambda i: (0, 0)),
            pl.BlockSpec((1, 128), lambda i: (0, 0)),
            pl.BlockSpec((128, 32), lambda i: (0, 0)),
            pl.BlockSpec((16, 32), lambda i: (0, 0)),
            pl.BlockSpec((1, 32), lambda i: (0, 0)),
        ],
        out_specs=[
            pl.BlockSpec((blk, 32), lambda i: (i, 0)),
            pl.BlockSpec((blk, 8), lambda i: (i, 0)),
        ],
        out_shape=[
            jax.ShapeDtypeStruct((NNODE, 32), jnp.float32),
            jax.ShapeDtypeStruct((NNODE, 8), jnp.float32),
        ],
    )(p0, p1, e0, e1, m0, m1, c0, c1, xl1, we1, att1, b1, w23, we23, att23)


def _combine2_body(q0_ref, q1_ref, xf_ref, es_ref, b2_ref, b3_ref, nl_ref, al_ref):
    s = q0_ref[...] + q1_ref[...]
    acc3 = s[:, :16]
    acc2 = s[:, 16:17]
    d2 = s[:, 17:18]
    d3 = s[:, 18:19]
    es = es_ref[...]
    ee2 = es[:, 0:1]
    ee3 = es[:, 1:2]
    xf = xf_ref[...]
    nl_ref[...] = (acc2 + ee2 * xf[:, 16:17]) / (d2 + ee2) + b2_ref[...]
    al_ref[...] = (acc3 + ee3 * xf[:, :16]) / (d3 + ee3) + b3_ref[...]


def _combine2(q0, q1, xf, es, b2, b3):
    blk = 400
    return pl.pallas_call(
        _combine2_body,
        grid=(NNODE // blk,),
        in_specs=[
            pl.BlockSpec((blk, 32), lambda i: (i, 0)),
            pl.BlockSpec((blk, 32), lambda i: (i, 0)),
            pl.BlockSpec((blk, 32), lambda i: (i, 0)),
            pl.BlockSpec((blk, 8), lambda i: (i, 0)),
            pl.BlockSpec((1, 1), lambda i: (0, 0)),
            pl.BlockSpec((1, 16), lambda i: (0, 0)),
        ],
        out_specs=[
            pl.BlockSpec((blk, 1), lambda i: (i, 0)),
            pl.BlockSpec((blk, 16), lambda i: (i, 0)),
        ],
        out_shape=[
            jax.ShapeDtypeStruct((NNODE, 1), jnp.float32),
            jax.ShapeDtypeStruct((NNODE, 16), jnp.float32),
        ],
    )(q0, q1, xf, es, b2, b3)


def _sc_mesh():
    return plsc.VectorSubcoreMesh(
        core_axis_name="c", subcore_axis_name="s", num_cores=2, num_subcores=16)


def _sc0_body(ei_hbm, ea_hbm, z_hbm, mout_hbm, cout_hbm,
              accm, accc, dstv, eav, ones, sem_d, sem_e, sem_sm, sem_sc):
    cid = lax.axis_index("c")
    sid = lax.axis_index("s")
    ebase = _wid_of(cid, sid) * EPW
    r0 = sid * RPT

    pltpu.sync_copy(z_hbm.at[pl.ds(r0, RPT), pl.ds(0, 16)],
                    accm.at[pl.ds(r0, RPT), :])
    pltpu.sync_copy(z_hbm.at[pl.ds(r0, RPT), pl.ds(16, 16)],
                    accc.at[pl.ds(r0, RPT), :])

    @pl.loop(0, CB2)
    def _init(i):
        ones[i, :] = jnp.where(lax.iota(jnp.int32, 16) == 0, 1.0, 0.0)

    plsc.subcore_barrier()

    def d_dst(ch, s):
        return pltpu.make_async_copy(
            ei_hbm.at[1, pl.ds(ebase + ch * CB2, CB2)], dstv.at[s], sem_d.at[s])

    def d_ea(ch, s):
        return pltpu.make_async_copy(
            ea_hbm.at[pl.ds(ebase + ch * CB2, CB2), :], eav.at[s], sem_e.at[s])

    def d_sm(s, b):
        return pltpu.make_async_copy(eav.at[s], accm.at[dstv.at[s]], sem_sm.at[b])

    def d_sc(s, b):
        return pltpu.make_async_copy(ones, accc.at[dstv.at[s]], sem_sc.at[b])

    pltpu.sync_copy(ei_hbm.at[1, pl.ds(ebase, CB2)], dstv.at[0])
    pltpu.sync_copy(ea_hbm.at[pl.ds(ebase, CB2), :], eav.at[0])
    d_dst(1, 1).start()
    d_ea(1, 1).start()

    @pl.loop(0, NCH2)
    def _chunk(ch):
        s4 = lax.rem(ch, 4)
        for ss in range(4):
            @pl.when(s4 == ss)
            def _():
                bb = ss & 1
                s1 = (ss + 1) % 4
                s2 = (ss + 2) % 4

                @pl.when(ch + 1 < NCH2)
                def _():
                    d_dst(ch + 1, s1).wait()
                    d_ea(ch + 1, s1).wait()

                @pl.when(ch >= 2)
                def _():
                    d_sm(s2, bb).wait()
                    d_sc(s2, bb).wait()

                @pl.when(ch + 2 < NCH2)
                def _():
                    d_dst(ch + 2, s2).start()
                    d_ea(ch + 2, s2).start()

                d_sm(ss, bb).start(add=True)
                d_sc(ss, bb).start(add=True)

    for chl in (NCH2 - 2, NCH2 - 1):
        d_sm(chl % 4, chl % 2).wait()
        d_sc(chl % 4, chl % 2).wait()

    plsc.subcore_barrier()
    pltpu.sync_copy(accm.at[pl.ds(r0, RPT), :], mout_hbm.at[cid, pl.ds(r0, RPT), :])
    pltpu.sync_copy(accc.at[pl.ds(r0, RPT), :], cout_hbm.at[cid, pl.ds(r0, RPT), :])


def _wid_of(cid, sid):
    return cid * 16 + sid


def _sc0_call(edge_index, edge_attr, z32):
    f = pl.kernel(
        _sc0_body,
        out_type=(jax.ShapeDtypeStruct((2, NNODE, 16), jnp.float32),
                  jax.ShapeDtypeStruct((2, NNODE, 16), jnp.float32)),
        mesh=_sc_mesh(),
        compiler_params=pltpu.CompilerParams(use_tc_tiling_on_sc=False),
        scratch_types=[
            pltpu.VMEM_SHARED((NNODE, 16), jnp.float32),
            pltpu.VMEM_SHARED((NNODE, 16), jnp.float32),
            pltpu.VMEM((4, CB2), jnp.int32),
            pltpu.VMEM((4, CB2, 16), jnp.float32),
            pltpu.VMEM((CB2, 16), jnp.float32),
            pltpu.SemaphoreType.DMA((4,)),
            pltpu.SemaphoreType.DMA((4,)),
            pltpu.SemaphoreType.DMA((2,)),
            pltpu.SemaphoreType.DMA((2,)),
        ],
    )
    return f(edge_index, edge_attr, z32)


def _sc1_body(xl1_hbm, g1_hbm, ei_hbm, att1_hbm, z128_hbm, z16_hbm,
              pout_hbm, eout_hbm,
              accp, acce, sd, comb, xdv, g1v, meta, att1v,
              sem_sd, sem_xs, sem_xd, sem_g, sem_scm, sem_sce):
    cid = lax.axis_index("c")
    sid = lax.axis_index("s")
    ebase = _wid_of(cid, sid) * EPW
    r0 = sid * RPT

    pltpu.sync_copy(z128_hbm.at[pl.ds(r0, RPT), :], accp.at[pl.ds(r0, RPT), :])
    pltpu.sync_copy(z16_hbm.at[pl.ds(r0, RPT), :], acce.at[pl.ds(r0, RPT), :])
    pltpu.sync_copy(att1_hbm, att1v)
    plsc.subcore_barrier()

    attc = [att1v[pl.ds(16 * k, 16)] for k in range(8)]
    io = lax.iota(jnp.int32, 16)
    l0 = jnp.where(io == 0, 1.0, 0.0).astype(jnp.float32)

    def d_sd(ch, s):
        return pltpu.make_async_copy(
            ei_hbm.at[:, pl.ds(ebase + ch * CB1, CB1)], sd.at[s], sem_sd.at[s])

    def d_xs(ch, s):
        return pltpu.make_async_copy(
            xl1_hbm.at[sd.at[s].at[0]], comb.at[s], sem_xs.at[s & 1])

    def d_xd(ch, s):
        return pltpu.make_async_copy(
            xl1_hbm.at[sd.at[s].at[1]], xdv.at[s & 1], sem_xd.at[s & 1])

    def d_g(ch, s):
        return pltpu.make_async_copy(
            g1_hbm.at[pl.ds(ebase + ch * CB1, CB1), :], g1v.at[s & 1], sem_g.at[s & 1])

    def d_scm(s):
        return pltpu.make_async_copy(
            comb.at[s], accp.at[sd.at[s].at[1]], sem_scm.at[s & 1])

    def d_sce(s):
        return pltpu.make_async_copy(
            meta.at[s & 1], acce.at[sd.at[s].at[1]], sem_sce.at[s & 1])

    pltpu.sync_copy(ei_hbm.at[:, pl.ds(ebase, CB1)], sd.at[0])
    d_xs(0, 0).start()
    d_xd(0, 0).start()
    d_g(0, 0).start()
    d_sd(1, 1).start()

    @pl.loop(0, NCH1)
    def _chunk(ch):
        s4 = lax.rem(ch, 4)
        for ss in range(4):
            @pl.when(s4 == ss)
            def _():
                bb = ss & 1
                s1 = (ss + 1) % 4
                s2 = (ss + 2) % 4

                @pl.when(ch + 1 < NCH1)
                def _():
                    d_sd(ch + 1, s1).wait()

                @pl.when(ch >= 2)
                def _():
                    d_scm(s2).wait()
                    d_sce(s2).wait()

                @pl.when(ch + 1 < NCH1)
                def _():
                    d_xs(ch + 1, s1).start()
                    d_xd(ch + 1, s1).start()
                    d_g(ch + 1, s1).start()

                @pl.when(ch + 2 < NCH1)
                def _():
                    d_sd(ch + 2, s2).start()

                d_xs(ch, ss).wait()
                d_xd(ch, ss).wait()
                d_g(ch, ss).wait()

                @pl.loop(0, CB1, unroll=4)
                def _edge(i):
                    xs = []
                    acc = jnp.zeros((16,), jnp.float32)
                    for k in range(8):
                        a = comb[ss, i, pl.ds(16 * k, 16)]
                        m = (a + xdv[bb, i, pl.ds(16 * k, 16)]
                             + g1v[bb, i, pl.ds(16 * k, 16)])
                        acc = acc + _lr(m) * attc[k]
                        xs.append(a)
                    ee = jnp.exp(_allsum16(acc))
                    for k in range(8):
                        comb[ss, i, pl.ds(16 * k, 16)] = xs[k] * ee
                    meta[bb, i, :] = ee * l0

                d_scm(ss).start(add=True)
                d_sce(ss).start(add=True)

    for chl in (NCH1 - 2, NCH1 - 1):
        d_scm(chl % 4).wait()
        d_sce(chl % 4).wait()

    plsc.subcore_barrier()
    pltpu.sync_copy(accp.at[pl.ds(r0, RPT), :], pout_hbm.at[cid, pl.ds(r0, RPT), :])
    pltpu.sync_copy(acce.at[pl.ds(r0, RPT), :], eout_hbm.at[cid, pl.ds(r0, RPT), :])


def _sc1_call(xl1, g1, edge_index, att1, z128, z16):
    f = pl.kernel(
        _sc1_body,
        out_type=(jax.ShapeDtypeStruct((2, NNODE, 128), jnp.float32),
                  jax.ShapeDtypeStruct((2, NNODE, 16), jnp.float32)),
        mesh=_sc_mesh(),
        compiler_params=pltpu.CompilerParams(use_tc_tiling_on_sc=False),
        scratch_types=[
            pltpu.VMEM_SHARED((NNODE, 128), jnp.float32),
            pltpu.VMEM_SHARED((NNODE, 16), jnp.float32),
            pltpu.VMEM((4, 2, CB1), jnp.int32),
            pltpu.VMEM((4, CB1, 128), jnp.float32),
            pltpu.VMEM((2, CB1, 128), jnp.float32),
            pltpu.VMEM((2, CB1, 128), jnp.float32),
            pltpu.VMEM((2, CB1, 16), jnp.float32),
            pltpu.VMEM((128,), jnp.float32),
            pltpu.SemaphoreType.DMA((4,)),
            pltpu.SemaphoreType.DMA((2,)),
            pltpu.SemaphoreType.DMA((2,)),
            pltpu.SemaphoreType.DMA((2,)),
            pltpu.SemaphoreType.DMA((2,)),
            pltpu.SemaphoreType.DMA((2,)),
        ],
    )
    return f(xl1, g1, edge_index, att1, z128, z16)


def _sc23_body(xf_hbm, g23_hbm, ei_hbm, att23_hbm, z32_hbm, out_hbm,
               acc_sh, sd, comb, xdv, g23v, att23v,
               sem_sd, sem_xs, sem_xd, sem_g, sem_sc):
    cid = lax.axis_index("c")
    sid = lax.axis_index("s")
    ebase = _wid_of(cid, sid) * EPW
    r0 = sid * RPT

    pltpu.sync_copy(z32_hbm.at[pl.ds(r0, RPT), :], acc_sh.at[pl.ds(r0, RPT), :])
    pltpu.sync_copy(att23_hbm, att23v)
    plsc.subcore_barrier()

    att3 = att23v[pl.ds(0, 16)]
    att2h = att23v[pl.ds(16, 16)]
    io = lax.iota(jnp.int32, 16)
    l0 = jnp.where(io == 0, 1.0, 0.0).astype(jnp.float32)
    l1 = jnp.where(io == 1, 1.0, 0.0).astype(jnp.float32)
    l2 = jnp.where(io == 2, 1.0, 0.0).astype(jnp.float32)

    def d_sd(ch, s):
        return pltpu.make_async_copy(
            ei_hbm.at[:, pl.ds(ebase + ch * CB2, CB2)], sd.at[s], sem_sd.at[s])

    def d_xs(ch, s):
        return pltpu.make_async_copy(
            xf_hbm.at[sd.at[s].at[0]], comb.at[s], sem_xs.at[s & 1])

    def d_xd(ch, s):
        return pltpu.make_async_copy(
            xf_hbm.at[sd.at[s].at[1]], xdv.at[s & 1], sem_xd.at[s & 1])

    def d_g(ch, s):
        return pltpu.make_async_copy(
            g23_hbm.at[pl.ds(ebase + ch * CB2, CB2), :], g23v.at[s & 1],
            sem_g.at[s & 1])

    def d_sc(s):
        return pltpu.make_async_copy(
            comb.at[s], acc_sh.at[sd.at[s].at[1]], sem_sc.at[s & 1])

    pltpu.sync_copy(ei_hbm.at[:, pl.ds(ebase, CB2)], sd.at[0])
    d_xs(0, 0).start()
    d_xd(0, 0).start()
    d_g(0, 0).start()
    d_sd(1, 1).start()

    @pl.loop(0, NCH2)
    def _chunk(ch):
        s4 = lax.rem(ch, 4)
        for ss in range(4):
            @pl.when(s4 == ss)
            def _():
                bb = ss & 1
                s1 = (ss + 1) % 4
                s2 = (ss + 2) % 4

                @pl.when(ch + 1 < NCH2)
                def _():
                    d_sd(ch + 1, s1).wait()

                @pl.when(ch >= 2)
                def _():
                    d_sc(s2).wait()

                @pl.when(ch + 1 < NCH2)
                def _():
                    d_xs(ch + 1, s1).start()
                    d_xd(ch + 1, s1).start()
                    d_g(ch + 1, s1).start()

                @pl.when(ch + 2 < NCH2)
                def _():
                    d_sd(ch + 2, s2).start()

                d_xs(ch, ss).wait()
                d_xd(ch, ss).wait()
                d_g(ch, ss).wait()

                @pl.loop(0, CB2, unroll=4)
                def _edge(i):
                    xs_lo = comb[ss, i, pl.ds(0, 16)]
                    xs_hi = comb[ss, i, pl.ds(16, 16)]
                    m3 = (xs_lo + xdv[bb, i, pl.ds(0, 16)]
                          + g23v[bb, i, pl.ds(0, 16)])
                    v2 = (xs_hi + xdv[bb, i, pl.ds(16, 16)]
                          + g23v[bb, i, pl.ds(16, 16)])
                    ee3 = jnp.exp(_allsum16(_lr(m3) * att3))
                    ee2 = jnp.exp(_allsum16(_lr(v2) * att2h))
                    comb[ss, i, pl.ds(0, 16)] = xs_lo * ee3
                    comb[ss, i, pl.ds(16, 16)] = (
                        ee2 * (xs_hi * l0 + l1) + ee3 * l2)

                d_sc(ss).start(add=True)

    for chl in (NCH2 - 2, NCH2 - 1):
        d_sc(chl % 4).wait()

    plsc.subcore_barrier()
    pltpu.sync_copy(acc_sh.at[pl.ds(r0, RPT), :], out_hbm.at[cid, pl.ds(r0, RPT), :])


def _sc23_call(xf, g23, edge_index, att23, z32):
    f = pl.kernel(
        _sc23_body,
        out_type=jax.ShapeDtypeStruct((2, NNODE, 32), jnp.float32),
        mesh=_sc_mesh(),
        compiler_params=pltpu.CompilerParams(use_tc_tiling_on_sc=False),
        scratch_types=[
            pltpu.VMEM_SHARED((NNODE, 32), jnp.float32),
            pltpu.VMEM((4, 2, CB2), jnp.int32),
            pltpu.VMEM((4, CB2, 32), jnp.float32),
            pltpu.VMEM((2, CB2, 32), jnp.float32),
            pltpu.VMEM((2, CB2, 32), jnp.float32),
            pltpu.VMEM((32,), jnp.float32),
            pltpu.SemaphoreType.DMA((4,)),
            pltpu.SemaphoreType.DMA((2,)),
            pltpu.SemaphoreType.DMA((2,)),
            pltpu.SemaphoreType.DMA((2,)),
            pltpu.SemaphoreType.DMA((2,)),
        ],
    )
    return f(xf, g23, edge_index, att23, z32)


def kernel(x, edge_index, edge_attr, W1, att1, We1, b1, W2, att2, We2, b2,
           W3, att3, We3, b3):
    f32 = jnp.float32

    we23 = jnp.concatenate([We3, We2, jnp.zeros((16, 15), f32)], axis=1)

    xl1 = _matmul(x, W1, 400)
    g1, g23 = _edge_proj(edge_attr, We1, we23)

    z32 = jnp.zeros((NNODE, 32), f32)
    z128 = jnp.zeros((NNODE, 128), f32)
    z16 = jnp.zeros((NNODE, 16), f32)

    mp, cp = _sc0_call(edge_index, edge_attr, z32)

    p, e = _sc1_call(xl1, g1, edge_index, att1, z128, z16)

    att23 = jnp.concatenate([att3, att2, jnp.zeros((15,), f32)]).reshape(1, 32)
    w23 = jnp.concatenate([W3, W2, jnp.zeros((128, 15), f32)], axis=1)
    xf, es = _combine1(p[0], p[1], e[0], e[1], mp[0], mp[1], cp[0], cp[1],
                       xl1, We1, att1.reshape(1, 128), b1.reshape(1, 128),
                       w23, we23, att23)

    q = _sc23_call(xf, g23, edge_index, att23.reshape(32), z32)

    nl, al = _combine2(q[0], q[1], xf, es, b2.reshape(1, 1), b3.reshape(1, 16))
    node_logits = nl[:, 0]
    action_logits = al

    node_sel = jax.random.categorical(jax.random.key(42), node_logits)
    node_lp = jax.nn.log_softmax(node_logits)[node_sel]
    alr = action_logits[node_sel, :]
    act_sel = jax.random.categorical(jax.random.key(43), alr)
    act_lp = jax.nn.log_softmax(alr)[act_sel]
    return (node_sel, act_sel, node_lp + act_lp)
